# baseline (speedup 1.0000x reference)
import functools

import jax
import jax.numpy as jnp
from jax import lax
from jax.experimental import pallas as pl
from jax.experimental.pallas import tpu as pltpu

S = 2048
D_MODEL = 1024
D_FF = 4096
N_EXPERT = 8
TOP_K = 2
ROW_TILE = 256


def _gelu_new(x):
    return 0.5 * x * (1.0 + jnp.tanh(jnp.sqrt(2.0 / jnp.pi) * (x + 0.044715 * jnp.power(x, 3.0))))


def _router_body(x_ref, wg_ref, logits_ref, wfull_ref):
    x = x_ref[...]
    wg = wg_ref[...]
    logits = lax.dot_general(x, wg, (((1,), (0,)), ((), ())),
                             preferred_element_type=jnp.float32)
    logits_ref[...] = logits
    m = jnp.max(logits, axis=1, keepdims=True)
    ex = jnp.exp(logits - m)
    probs = ex / jnp.sum(ex, axis=1, keepdims=True)
    iota8 = lax.broadcasted_iota(jnp.int32, (S, N_EXPERT), 1)
    m1 = jnp.max(probs, axis=1, keepdims=True)
    e1 = jnp.min(jnp.where(probs == m1, iota8, N_EXPERT), axis=1, keepdims=True)
    masked = jnp.where(iota8 == e1, -jnp.inf, probs)
    m2 = jnp.max(masked, axis=1, keepdims=True)
    e2 = jnp.min(jnp.where(masked == m2, iota8, N_EXPERT), axis=1, keepdims=True)
    denom = m1 + m2
    wfull_ref[...] = (jnp.where(iota8 == e1, m1 / denom, 0.0)
                      + jnp.where(iota8 == e2, m2 / denom, 0.0))


def _ffn_body(x_ref, wfc_ref, bfc_ref, wproj_ref, bproj_ref, wfull_ref, out_ref):
    e = pl.program_id(1)

    @pl.when(e == 0)
    def _():
        out_ref[...] = jnp.zeros_like(out_ref)

    x = x_ref[...].astype(jnp.bfloat16)
    h = lax.dot_general(x, wfc_ref[0], (((1,), (0,)), ((), ())),
                        preferred_element_type=jnp.float32)
    h = _gelu_new(h + bfc_ref[0])
    y = lax.dot_general(h.astype(jnp.bfloat16), wproj_ref[0], (((1,), (0,)), ((), ())),
                        preferred_element_type=jnp.float32)
    y = y + bproj_ref[0]
    iota8 = lax.broadcasted_iota(jnp.int32, (ROW_TILE, N_EXPERT), 1)
    w_col = jnp.sum(jnp.where(iota8 == e, wfull_ref[...], 0.0), axis=1, keepdims=True)
    out_ref[...] += w_col * y


def kernel(hidden_states, W_g, c_fc_w, c_fc_b, c_proj_w, c_proj_b):
    b, s, d = hidden_states.shape
    hs = hidden_states.reshape(s, d)

    logits, wfull = pl.pallas_call(
        _router_body,
        out_shape=(
            jax.ShapeDtypeStruct((S, N_EXPERT), jnp.float32),
            jax.ShapeDtypeStruct((S, N_EXPERT), jnp.float32),
        ),
    )(hs, W_g)

    wfc = c_fc_w.astype(jnp.bfloat16)
    wproj = c_proj_w.astype(jnp.bfloat16)

    n_rt = S // ROW_TILE
    out = pl.pallas_call(
        _ffn_body,
        grid=(n_rt, N_EXPERT),
        in_specs=[
            pl.BlockSpec((ROW_TILE, D_MODEL), lambda rt, e: (rt, 0)),
            pl.BlockSpec((1, D_MODEL, D_FF), lambda rt, e: (e, 0, 0)),
            pl.BlockSpec((1, 1, D_FF), lambda rt, e: (e, 0, 0)),
            pl.BlockSpec((1, D_FF, D_MODEL), lambda rt, e: (e, 0, 0)),
            pl.BlockSpec((1, 1, D_MODEL), lambda rt, e: (e, 0, 0)),
            pl.BlockSpec((ROW_TILE, N_EXPERT), lambda rt, e: (rt, 0)),
        ],
        out_specs=pl.BlockSpec((ROW_TILE, D_MODEL), lambda rt, e: (rt, 0)),
        out_shape=jax.ShapeDtypeStruct((S, D_MODEL), jnp.float32),
    )(hs, wfc, c_fc_b.reshape(N_EXPERT, 1, D_FF), wproj,
      c_proj_b.reshape(N_EXPERT, 1, D_MODEL), wfull)

    return (out.reshape(b, s, d), logits)


# speedup vs baseline: 1.5668x; 1.5668x over previous
import functools

import jax
import jax.numpy as jnp
from jax import lax
from jax.experimental import pallas as pl
from jax.experimental.pallas import tpu as pltpu
from jax.experimental.pallas import tpu_sc as plsc

S = 2048
D_MODEL = 1024
D_FF = 4096
N_EXPERT = 8
TOP_K = 2
ROW_TILE = 256
BLK = 256
M = S * TOP_K + N_EXPERT * ROW_TILE
NT = M // ROW_TILE
NW = 32


def _gelu_new(x):
    return 0.5 * x * (1.0 + jnp.tanh(jnp.sqrt(2.0 / jnp.pi) * (x + 0.044715 * jnp.power(x, 3.0))))


def _router_body(x_ref, wg_ref, logits_ref, posk_ref, wk_ref, te_ref,
                 oh0_s, oh1_s, r0_s, r1_s):
    x = x_ref[...]
    logits = lax.dot_general(x, wg_ref[...], (((1,), (0,)), ((), ())),
                             preferred_element_type=jnp.float32)
    logits_ref[...] = logits
    m = jnp.max(logits, axis=1, keepdims=True)
    ex = jnp.exp(logits - m)
    probs = ex / jnp.sum(ex, axis=1, keepdims=True)
    iota8 = lax.broadcasted_iota(jnp.int32, (S, N_EXPERT), 1)
    m1 = jnp.max(probs, axis=1, keepdims=True)
    e1 = jnp.min(jnp.where(probs == m1, iota8, N_EXPERT), axis=1, keepdims=True)
    masked = jnp.where(iota8 == e1, -jnp.inf, probs)
    m2 = jnp.max(masked, axis=1, keepdims=True)
    e2 = jnp.min(jnp.where(masked == m2, iota8, N_EXPERT), axis=1, keepdims=True)
    denom = m1 + m2
    w1 = m1 / denom
    w2 = m2 / denom

    oh0_s[...] = (iota8 == e1).astype(jnp.float32)
    oh1_s[...] = (iota8 == e2).astype(jnp.float32)

    rb = (lax.broadcasted_iota(jnp.int32, (BLK, BLK), 0)
          > lax.broadcasted_iota(jnp.int32, (BLK, BLK), 1)).astype(jnp.float32)

    def step(i, carry):
        c0, c1 = carry
        b0 = oh0_s[pl.ds(i * BLK, BLK), :]
        b1 = oh1_s[pl.ds(i * BLK, BLK), :]
        r0_s[pl.ds(i * BLK, BLK), :] = lax.dot_general(
            rb, b0, (((1,), (0,)), ((), ())), preferred_element_type=jnp.float32) + c0
        r1_s[pl.ds(i * BLK, BLK), :] = lax.dot_general(
            rb, b1, (((1,), (0,)), ((), ())), preferred_element_type=jnp.float32) + c1
        return (c0 + jnp.sum(b0, axis=0, keepdims=True),
                c1 + jnp.sum(b1, axis=0, keepdims=True))

    c0, c1 = lax.fori_loop(0, S // BLK, step,
                           (jnp.zeros((1, N_EXPERT), jnp.float32),
                            jnp.zeros((1, N_EXPERT), jnp.float32)))
    counts = c0 + c1
    pc = jnp.ceil(counts / ROW_TILE) * ROW_TILE
    tri8 = (lax.broadcasted_iota(jnp.int32, (N_EXPERT, N_EXPERT), 0)
            < lax.broadcasted_iota(jnp.int32, (N_EXPERT, N_EXPERT), 1)).astype(jnp.float32)
    po = lax.dot_general(pc, tri8, (((1,), (0,)), ((), ())),
                         preferred_element_type=jnp.float32)

    oh0 = oh0_s[...]
    oh1 = oh1_s[...]

    def sel(mat, oh):
        return jnp.sum(mat * oh, axis=1, keepdims=True)

    pos0 = sel(po, oh0) + sel(r0_s[...], oh0)
    pos1 = sel(po, oh1) + sel(c0, oh1) + sel(r1_s[...], oh1)
    posk = (jnp.where(iota8 == 0, pos0, 0.0)
            + jnp.where(iota8 == 1, pos1, 0.0))
    posk_ref[...] = jnp.transpose(posk).astype(jnp.int32)
    iota32 = lax.broadcasted_iota(jnp.int32, (S, 32), 1)
    wk_ref[...] = jnp.where(iota32 < 16, w1, w2)

    iota128 = lax.broadcasted_iota(jnp.int32, (1, 128), 1)
    ti = iota128.astype(jnp.float32) * ROW_TILE
    iota_e = lax.broadcasted_iota(jnp.int32, (1, N_EXPERT), 1)
    te = jnp.zeros((1, 128), jnp.float32)
    last_e = jnp.float32(0.0)
    for e in range(N_EXPERT):
        oh_e = (iota_e == e).astype(jnp.float32)
        po_e = jnp.sum(po * oh_e)
        pc_e = jnp.sum(pc * oh_e)
        te += (ti >= po_e).astype(jnp.float32)
        last_e = jnp.maximum(last_e, jnp.where(pc_e > 0, jnp.float32(e), 0.0))
    te = jnp.minimum(te - 1.0, last_e)
    te = jnp.maximum(te, 0.0)
    n_used = jnp.sum(pc) / ROW_TILE
    te = jnp.where(iota128 == 127, n_used, te)
    te_ref[...] = te.astype(jnp.int32)


@functools.lru_cache(maxsize=None)
def _get_dispatch():
    mesh = plsc.VectorSubcoreMesh(core_axis_name="c", subcore_axis_name="s")

    @functools.partial(
        pl.kernel, mesh=mesh,
        out_type=jax.ShapeDtypeStruct((M, D_MODEL), jnp.float32),
        scratch_types=[
            pltpu.VMEM((64,), jnp.int32),
            pltpu.VMEM((64, D_MODEL), jnp.float32),
            pltpu.SemaphoreType.DMA,
        ],
    )
    def _dispatch(hs_hbm, posk_hbm, xd_hbm, idx_v, rows_v, sem):
        wid = lax.axis_index("s") * 2 + lax.axis_index("c")
        k = wid // 16
        t0 = (wid % 16) * 128
        for j in range(2):
            tj = t0 + j * 64
            pltpu.sync_copy(posk_hbm.at[pl.ds(k * S + tj, 64)], idx_v)
            pltpu.sync_copy(hs_hbm.at[pl.ds(tj, 64)], rows_v)
            pltpu.async_copy(rows_v, xd_hbm.at[idx_v], sem).wait()

    return _dispatch


def _ffn_body(te_ref, x_ref, wfc_ref, bfc_ref, wproj_ref, bproj_ref, y_ref):
    i = pl.program_id(0)

    @pl.when(i < te_ref[127])
    def _():
        x = x_ref[...].astype(jnp.bfloat16)
        h = lax.dot_general(x, wfc_ref[0], (((1,), (0,)), ((), ())),
                            preferred_element_type=jnp.float32)
        h = _gelu_new(h + bfc_ref[0])
        y = lax.dot_general(h.astype(jnp.bfloat16), wproj_ref[0], (((1,), (0,)), ((), ())),
                            preferred_element_type=jnp.float32)
        y_ref[...] = y + bproj_ref[0]


@functools.lru_cache(maxsize=None)
def _get_combine():
    mesh = plsc.VectorSubcoreMesh(core_axis_name="c", subcore_axis_name="s")

    @functools.partial(
        pl.kernel, mesh=mesh,
        out_type=jax.ShapeDtypeStruct((S, D_MODEL), jnp.float32),
        scratch_types=[
            pltpu.VMEM((32, 32), jnp.float32),
            pltpu.VMEM((32,), jnp.int32),
            pltpu.VMEM((32,), jnp.int32),
            pltpu.VMEM((32, D_MODEL), jnp.float32),
            pltpu.VMEM((32, D_MODEL), jnp.float32),
            pltpu.VMEM((32, D_MODEL), jnp.float32),
            pltpu.SemaphoreType.DMA,
        ],
    )
    def _combine(y_hbm, posk_hbm, wk_hbm, out_hbm,
                 wchunk_v, idx0_v, idx1_v, y0_v, y1_v, o_v, sem):
        wid = lax.axis_index("s") * 2 + lax.axis_index("c")
        for j in range(2):
            t0 = wid * 64 + j * 32
            pltpu.sync_copy(posk_hbm.at[pl.ds(t0, 32)], idx0_v)
            pltpu.sync_copy(posk_hbm.at[pl.ds(S + t0, 32)], idx1_v)
            pltpu.sync_copy(wk_hbm.at[pl.ds(t0, 32)], wchunk_v)
            pltpu.async_copy(y_hbm.at[idx0_v], y0_v, sem).wait()
            pltpu.async_copy(y_hbm.at[idx1_v], y1_v, sem).wait()

            def tok(i, carry):
                w0 = wchunk_v[i, pl.ds(0, 16)]
                w1 = wchunk_v[i, pl.ds(16, 16)]
                for c in range(D_MODEL // 16):
                    o_v[i, pl.ds(c * 16, 16)] = (w0 * y0_v[i, pl.ds(c * 16, 16)]
                                                 + w1 * y1_v[i, pl.ds(c * 16, 16)])
                return carry

            lax.fori_loop(0, 32, tok, 0)
            pltpu.sync_copy(o_v, out_hbm.at[pl.ds(t0, 32)])

    return _combine


def kernel(hidden_states, W_g, c_fc_w, c_fc_b, c_proj_w, c_proj_b):
    b, s, d = hidden_states.shape
    hs = hidden_states.reshape(s, d)

    logits, posk, wk, te = pl.pallas_call(
        _router_body,
        out_shape=(
            jax.ShapeDtypeStruct((S, N_EXPERT), jnp.float32),
            jax.ShapeDtypeStruct((N_EXPERT, S), jnp.int32),
            jax.ShapeDtypeStruct((S, 32), jnp.float32),
            jax.ShapeDtypeStruct((1, 128), jnp.int32),
        ),
        scratch_shapes=[pltpu.VMEM((S, N_EXPERT), jnp.float32)] * 4,
    )(hs, W_g)

    posk_flat = posk[:TOP_K].reshape(TOP_K * S)
    x_disp = _get_dispatch()(hs, posk_flat)

    wfc = c_fc_w.astype(jnp.bfloat16)
    wproj = c_proj_w.astype(jnp.bfloat16)

    y = pl.pallas_call(
        _ffn_body,
        grid_spec=pltpu.PrefetchScalarGridSpec(
            num_scalar_prefetch=1,
            grid=(NT,),
            in_specs=[
                pl.BlockSpec((ROW_TILE, D_MODEL), lambda i, te: (i, 0)),
                pl.BlockSpec((1, D_MODEL, D_FF), lambda i, te: (te[i], 0, 0)),
                pl.BlockSpec((1, 1, D_FF), lambda i, te: (te[i], 0, 0)),
                pl.BlockSpec((1, D_FF, D_MODEL), lambda i, te: (te[i], 0, 0)),
                pl.BlockSpec((1, 1, D_MODEL), lambda i, te: (te[i], 0, 0)),
            ],
            out_specs=pl.BlockSpec((ROW_TILE, D_MODEL), lambda i, te: (i, 0)),
        ),
        out_shape=jax.ShapeDtypeStruct((M, D_MODEL), jnp.float32),
    )(te.reshape(128), x_disp, wfc, c_fc_b.reshape(N_EXPERT, 1, D_FF),
      wproj, c_proj_b.reshape(N_EXPERT, 1, D_MODEL))

    final = _get_combine()(y, posk_flat, wk)
    return (final.reshape(b, s, d), logits)
